# baseline (speedup 1.0000x reference)
import functools

import jax
import jax.numpy as jnp
from jax import lax
from jax.experimental import pallas as pl
from jax.experimental.pallas import tpu as pltpu
from jax.experimental.pallas import tpu_sc as plsc

_NC = 2
_NS = 16
_NW = _NC * _NS

_PB = 4096
_HP = 123 * _PB


def _pack_table(Wt):
    D, V = Wt.shape
    G = _HP // _PB
    last_b = (V - 1) // _PB

    def body(a_ref, b_ref, out_ref):
        eye = (lax.broadcasted_iota(jnp.int32, (D, D), 0)
               == lax.broadcasted_iota(jnp.int32, (D, D), 1)).astype(jnp.float32)
        dn = (((0,), (0,)), ((), ()))
        out_ref[:, 0:D] = lax.dot_general(
            a_ref[...], eye, dn, preferred_element_type=jnp.float32)
        out_ref[:, D:2 * D] = lax.dot_general(
            b_ref[...], eye, dn, preferred_element_type=jnp.float32)

    return pl.pallas_call(
        body,
        grid=(G,),
        in_specs=[
            pl.BlockSpec((D, _PB), lambda g: (0, g)),
            pl.BlockSpec((D, _PB), lambda g: (0, jnp.minimum(g + G, last_b))),
        ],
        out_specs=pl.BlockSpec((_PB, 2 * D), lambda g: (g, 0)),
        out_shape=jax.ShapeDtypeStruct((_HP, 2 * D), jnp.float32),
    )(Wt, Wt)


def _emb_call(n_chunks, chunk, D, B, S, idx, table):
    mesh = plsc.VectorSubcoreMesh(core_axis_name="c", subcore_axis_name="s")
    N = _NW * n_chunks * chunk
    assert N == B * S
    DQ = D // 8
    CB = B // chunk

    @functools.partial(
        pl.kernel,
        mesh=mesh,
        compiler_params=pltpu.CompilerParams(
            use_tc_tiling_on_sc=False, needs_layout_passes=False),
        out_type=jax.ShapeDtypeStruct((S, DQ, CB, 8, chunk), jnp.float32),
        scratch_types=[
            pltpu.VMEM((n_chunks, chunk), jnp.int32),
            pltpu.VMEM((chunk, D), jnp.float32),
            pltpu.VMEM((chunk, D), jnp.float32),
            pltpu.VMEM((DQ, 8, chunk + 1), jnp.float32),
            pltpu.VMEM((DQ, 8, chunk + 1), jnp.float32),
            pltpu.SemaphoreType.DMA,
            pltpu.SemaphoreType.DMA,
            pltpu.SemaphoreType.DMA,
        ],
    )
    def emb(idx_hbm, table_hbm, out_hbm, idx_v, rows0, rows1, tr0, tr1,
            isem, gsem, osem):
        wid = lax.axis_index("s") * _NC + lax.axis_index("c")
        base = wid * (n_chunks * chunk)
        pltpu.async_copy(idx_hbm.at[wid], idx_v, isem).wait()

        iot = lax.iota(jnp.int32, 16)

        def gather(j, rows):
            return pltpu.async_copy(table_hbm.at[idx_v.at[j]], rows, gsem)

        def out_slice(j):
            n0 = base + j * chunk
            s = n0 // B
            c = (n0 - s * B) // chunk
            return out_hbm.at[s, :, c]

        def put(j, tr):
            return pltpu.async_copy(
                tr.at[:, :, pl.ds(0, chunk)], out_slice(j), osem)

        dgv = [iot + g * 16 for g in range(D // 16)]
        dqv = [v // 8 for v in dgv]
        drv = [v % 8 for v in dgv]

        def transpose(rows, tr):
            for l0 in range(0, chunk, 8):
                xs = []
                for l in range(l0, l0 + 8):
                    for g in range(D // 16):
                        xs.append((l, g, rows[l, pl.ds(g * 16, 16)]))
                for l, g, x in xs:
                    col = jnp.full((16,), l, jnp.int32)
                    plsc.store_scatter(tr, [dqv[g], drv[g], col], x)

        rbufs = (rows0, rows1)
        tbufs = (tr0, tr1)

        gather(0, rows0).wait()

        def pair_body(i, _):
            j0 = 2 * i
            for half in range(2):
                j = j0 + half
                rows, tr = rbufs[half], tbufs[half]
                nrows = rbufs[1 - half]

                @pl.when(j + 1 < n_chunks)
                def _():
                    gather(j + 1, nrows)

                transpose(rows, tr)

                @pl.when(j >= 1)
                def _():
                    pltpu.make_async_copy(
                        tbufs[1 - half].at[:, :, pl.ds(0, chunk)],
                        out_slice(j - 1), osem).wait()

                put(j, tr)

                @pl.when(j + 1 < n_chunks)
                def _():
                    pltpu.make_async_copy(
                        table_hbm.at[idx_v.at[j]], nrows, gsem).wait()
            return 0

        lax.fori_loop(0, n_chunks // 2, pair_body, 0, unroll=False)
        pltpu.make_async_copy(
            tbufs[(n_chunks - 1) % 2].at[:, :, pl.ds(0, chunk)],
            out_slice(n_chunks - 1), osem).wait()

    return emb(idx, table)


def kernel(tokens, W_E):
    B, S = tokens.shape
    V, D = W_E.shape
    N = B * S
    chunk = 128
    n_chunks = N // (_NW * chunk)
    assert N == _NW * n_chunks * chunk

    t32 = tokens.T.astype(jnp.int32)
    idx = jnp.where(t32 < _HP, 2 * t32, 2 * (t32 - _HP) + 1)
    idx = idx.reshape(_NW, n_chunks, chunk)

    table_lin = _pack_table(W_E.T).reshape(2 * _HP, D)
    out5 = _emb_call(n_chunks, chunk, D, B, S, idx, table_lin)
    return out5.transpose(2, 4, 0, 1, 3).reshape(B, S, D)


# speedup vs baseline: 1.1070x; 1.1070x over previous
import functools

import jax
import jax.numpy as jnp
from jax import lax
from jax.experimental import pallas as pl
from jax.experimental.pallas import tpu as pltpu
from jax.experimental.pallas import tpu_sc as plsc

_NC = 2
_NS = 16
_NW = _NC * _NS

_PB = 4096
_HP = 123 * _PB


def _pack_table(Wt):
    D, V = Wt.shape
    G = _HP // _PB
    last_b = (V - 1) // _PB

    def body(a_ref, b_ref, out_ref):
        eye = (lax.broadcasted_iota(jnp.int32, (D, D), 0)
               == lax.broadcasted_iota(jnp.int32, (D, D), 1)).astype(jnp.float32)
        dn = (((0,), (0,)), ((), ()))
        out_ref[:, 0:D] = lax.dot_general(
            a_ref[...], eye, dn, preferred_element_type=jnp.float32)
        out_ref[:, D:2 * D] = lax.dot_general(
            b_ref[...], eye, dn, preferred_element_type=jnp.float32)

    return pl.pallas_call(
        body,
        grid=(G,),
        in_specs=[
            pl.BlockSpec((D, _PB), lambda g: (0, g)),
            pl.BlockSpec((D, _PB), lambda g: (0, jnp.minimum(g + G, last_b))),
        ],
        out_specs=pl.BlockSpec((_PB, 2 * D), lambda g: (g, 0)),
        out_shape=jax.ShapeDtypeStruct((_HP, 2 * D), jnp.float32),
    )(Wt, Wt)


def _emb_call(n_chunks, chunk, D, B, S, idx, table):
    mesh = plsc.VectorSubcoreMesh(core_axis_name="c", subcore_axis_name="s")
    N = _NW * n_chunks * chunk
    assert N == B * S
    DQ = D // 8
    CB = B // chunk

    @functools.partial(
        pl.kernel,
        mesh=mesh,
        compiler_params=pltpu.CompilerParams(
            use_tc_tiling_on_sc=False, needs_layout_passes=False),
        out_type=jax.ShapeDtypeStruct((S, DQ, CB, 8, chunk), jnp.float32),
        scratch_types=[
            pltpu.VMEM((n_chunks, chunk), jnp.int32),
            pltpu.VMEM((chunk, D), jnp.float32),
            pltpu.VMEM((chunk, D), jnp.float32),
            pltpu.VMEM((DQ, 8, chunk + 1), jnp.float32),
            pltpu.VMEM((DQ, 8, chunk + 1), jnp.float32),
            pltpu.SemaphoreType.DMA,
            pltpu.SemaphoreType.DMA,
            pltpu.SemaphoreType.DMA,
        ],
    )
    def emb(idx_hbm, table_hbm, out_hbm, idx_v, rows0, rows1, tr0, tr1,
            isem, gsem, osem):
        wid = lax.axis_index("s") * _NC + lax.axis_index("c")
        base = wid * (n_chunks * chunk)
        pltpu.async_copy(idx_hbm.at[wid], idx_v, isem).wait()

        iot = lax.iota(jnp.int32, 16)

        def gather(j, rows):
            return pltpu.async_copy(table_hbm.at[idx_v.at[j]], rows, gsem)

        def out_slice(j):
            n0 = base + j * chunk
            s = n0 // B
            c = (n0 - s * B) // chunk
            return out_hbm.at[s, :, c]

        def put(j, tr):
            return pltpu.async_copy(
                tr.at[:, :, pl.ds(0, chunk)], out_slice(j), osem)

        dgv = [iot + g * 16 for g in range(D // 16)]
        dqv = [v // 8 for v in dgv]
        drv = [v % 8 for v in dgv]

        def transpose(rows, tr):
            nb = 2
            for l0 in range(0, chunk, nb):
                xs = []
                for l in range(l0, l0 + nb):
                    for g in range(D // 16):
                        xs.append((l, g, rows[l, pl.ds(g * 16, 16)]))
                for l, g, x in xs:
                    col = jnp.full((16,), l, jnp.int32)
                    plsc.store_scatter(tr, [dqv[g], drv[g], col], x)

        rbufs = (rows0, rows1)
        tbufs = (tr0, tr1)

        gather(0, rows0).wait()

        def pair_body(i, _):
            j0 = 2 * i
            for half in range(2):
                j = j0 + half
                rows, tr = rbufs[half], tbufs[half]
                nrows = rbufs[1 - half]

                @pl.when(j + 1 < n_chunks)
                def _():
                    gather(j + 1, nrows)

                transpose(rows, tr)

                @pl.when(j >= 1)
                def _():
                    pltpu.make_async_copy(
                        tbufs[1 - half].at[:, :, pl.ds(0, chunk)],
                        out_slice(j - 1), osem).wait()

                put(j, tr)

                @pl.when(j + 1 < n_chunks)
                def _():
                    pltpu.make_async_copy(
                        table_hbm.at[idx_v.at[j]], nrows, gsem).wait()
            return 0

        lax.fori_loop(0, n_chunks // 2, pair_body, 0, unroll=False)
        pltpu.make_async_copy(
            tbufs[(n_chunks - 1) % 2].at[:, :, pl.ds(0, chunk)],
            out_slice(n_chunks - 1), osem).wait()

    return emb(idx, table)


def kernel(tokens, W_E):
    B, S = tokens.shape
    V, D = W_E.shape
    N = B * S
    chunk = 128
    n_chunks = N // (_NW * chunk)
    assert N == _NW * n_chunks * chunk

    t32 = tokens.T.astype(jnp.int32)
    idx = jnp.where(t32 < _HP, 2 * t32, 2 * (t32 - _HP) + 1)
    idx = idx.reshape(_NW, n_chunks, chunk)

    table_lin = _pack_table(W_E.T).reshape(2 * _HP, D)
    out5 = _emb_call(n_chunks, chunk, D, B, S, idx, table_lin)
    return out5.transpose(2, 4, 0, 1, 3).reshape(B, S, D)
